# baseline (speedup 1.0000x reference)
import functools

import jax
import jax.numpy as jnp
from jax.experimental import pallas as pl
from jax.experimental.pallas import tpu as pltpu


def _body(x_ref, c_ref, m_ref,
          in_w_ref, sw_ref, rs_w_ref, in_b_ref, rs_b_ref,
          o_ref,
          ws_ref, wz_s, wrs_s, bz_s, acts_ref,
          *, seqs, chans, stack, taps, folds, dilation_rate):
    G, C, S, K, F = seqs, chans, stack, taps, folds
    R = G * C
    E = 16
    T = x_ref.shape[-1]

    @pl.when(pl.program_id(0) == 0)
    def _prep():
        p_row = jax.lax.broadcasted_iota(jnp.int32, (R, C), 0) % C
        p_col = jax.lax.broadcasted_iota(jnp.int32, (R, C), 1)
        P = (p_row == p_col).astype(jnp.float32)
        q_row = jax.lax.broadcasted_iota(jnp.int32, (C, R), 0)
        q_col = jax.lax.broadcasted_iota(jnp.int32, (C, R), 1) % C
        Q = (q_row == q_col).astype(jnp.float32)
        blk = (jax.lax.broadcasted_iota(jnp.int32, (R, R), 0) // C ==
               jax.lax.broadcasted_iota(jnp.int32, (R, R), 1) // C)

        def bd(w):
            tiled = jnp.dot(jnp.dot(P, w, preferred_element_type=jnp.float32),
                            Q, preferred_element_type=jnp.float32)
            return jnp.where(blk, tiled, 0.0).astype(jnp.bfloat16)

        def tile_b(b):
            return jnp.dot(P, b, preferred_element_type=jnp.float32)

        acts_ref[R:R + E, :] = jnp.zeros((E, acts_ref.shape[-1]),
                                         jnp.bfloat16)
        acts_ref[R:R + 2, :] = jnp.ones((2, acts_ref.shape[-1]),
                                        jnp.bfloat16)

        for i in range(S):
            for h in range(2):
                r0, r1 = h * R, (h + 1) * R
                for k in range(K):
                    wz_s[i, r0:r1, k * R:(k + 1) * R] = bd(
                        in_w_ref[i, k, h * C:(h + 1) * C, :])
                wz_s[i, r0:r1, K * R:(K + 1) * R] = bd(
                    sw_ref[i, h * C:(h + 1) * C, :])
                wrs_s[i, r0:r1, 0:R] = bd(rs_w_ref[i, h * C:(h + 1) * C, :])
                wrs_s[i, r0:r1, R:R + E] = jnp.zeros((R, E), jnp.bfloat16)
                b32 = tile_b(rs_b_ref[i, h * C:(h + 1) * C, :])
                b_hi = b32.astype(jnp.bfloat16)
                wrs_s[i, r0:r1, R:R + 1] = b_hi
                wrs_s[i, r0:r1, R + 1:R + 2] = (
                    b32 - b_hi.astype(jnp.float32)).astype(jnp.bfloat16)
                bz_s[i, r0:r1, :] = tile_b(in_b_ref[i, h * C:(h + 1) * C, :])

    xs = x_ref[...]
    ms = m_ref[...]
    lane = jax.lax.broadcasted_iota(jnp.int32, (R, T), 1)

    xv = [xs[j * G:(j + 1) * G].reshape(R, T) for j in range(F)]
    mask_b = [
        jnp.broadcast_to(ms[j * G:(j + 1) * G], (G, C, T)).reshape(R, T)
        for j in range(F)]

    cs = c_ref[...]
    for j in range(F):
        ws_ref[3 * R:4 * R, j * T:(j + 1) * T] = (
            cs[j * G:(j + 1) * G].reshape(R, T).astype(jnp.bfloat16))

    skip = None
    for i in range(S):
        d = dilation_rate ** i
        for j in range(F):
            cols = slice(j * T, (j + 1) * T)
            ws_ref[0:R, cols] = jnp.where(
                lane >= d, pltpu.roll(xv[j], d, axis=1),
                0.0).astype(jnp.bfloat16)
            ws_ref[R:2 * R, cols] = xv[j].astype(jnp.bfloat16)
            ws_ref[2 * R:3 * R, cols] = jnp.where(
                lane < T - d, pltpu.roll(xv[j], T - d, axis=1),
                0.0).astype(jnp.bfloat16)
        z = (jnp.dot(wz_s[i], ws_ref[...], preferred_element_type=jnp.float32)
             + bz_s[i])
        acts_ref[0:R, :] = (jnp.tanh(z[0:R])
                            * jax.nn.sigmoid(z[R:2 * R])).astype(jnp.bfloat16)
        ro = jnp.dot(wrs_s[i], acts_ref[...],
                     preferred_element_type=jnp.float32)
        for j in range(F):
            xv[j] = (xv[j] + ro[0:R, j * T:(j + 1) * T]) * mask_b[j]
        skip = ro[R:2 * R] if i == 0 else skip + ro[R:2 * R]

    for j in range(F):
        o_ref[j * G:(j + 1) * G] = (
            skip[:, j * T:(j + 1) * T] * mask_b[j]).reshape(G, C, T).astype(
                o_ref.dtype)


def kernel(x, conditions, float_masks, style_w, in_b, in_w, rs_w, rs_b):
    B, C, T = x.shape
    Cs = conditions.shape[1]
    S, K = in_w.shape[0], in_w.shape[1]
    assert Cs == C
    dilation_rate = 2

    G, F = 16, 8
    while F > 1 and B % (G * F):
        F //= 2
    while B % (G * F):
        G //= 2
    R = G * C
    num_blocks = B // (G * F)

    sw = style_w.reshape(S, 2 * C, Cs)

    body = functools.partial(
        _body, seqs=G, chans=C, stack=S, taps=K, folds=F,
        dilation_rate=dilation_rate)

    const = lambda *shape: (shape, lambda b: (0,) * len(shape))

    out = pl.pallas_call(
        body,
        out_shape=jax.ShapeDtypeStruct((B, C, T), jnp.float32),
        grid=(num_blocks,),
        in_specs=[
            pl.BlockSpec((G * F, C, T), lambda b: (b, 0, 0)),
            pl.BlockSpec((G * F, C, T), lambda b: (b, 0, 0)),
            pl.BlockSpec((G * F, 1, T), lambda b: (b, 0, 0)),
            pl.BlockSpec(*const(S, K, 2 * C, C)),
            pl.BlockSpec(*const(S, 2 * C, Cs)),
            pl.BlockSpec(*const(S, 2 * C, C)),
            pl.BlockSpec(*const(S, 2 * C, 1)),
            pl.BlockSpec(*const(S, 2 * C, 1)),
        ],
        out_specs=pl.BlockSpec((G * F, C, T), lambda b: (b, 0, 0)),
        scratch_shapes=[
            pltpu.VMEM(((K + 1) * R, F * T), jnp.bfloat16),
            pltpu.VMEM((S, 2 * R, (K + 1) * R), jnp.bfloat16),
            pltpu.VMEM((S, 2 * R, R + 16), jnp.bfloat16),
            pltpu.VMEM((S, 2 * R, 1), jnp.float32),
            pltpu.VMEM((R + 16, F * T), jnp.bfloat16),
        ],
        compiler_params=pltpu.CompilerParams(
            dimension_semantics=("arbitrary",),
            vmem_limit_bytes=56 * 1024 * 1024),
    )(x, conditions, float_masks, in_w, sw, rs_w, in_b, rs_b)

    return out


# speedup vs baseline: 1.0186x; 1.0186x over previous
import functools

import jax
import jax.numpy as jnp
from jax.experimental import pallas as pl
from jax.experimental.pallas import tpu as pltpu


def _body(x_ref, c_ref, m_ref,
          in_w_ref, sw_ref, rs_w_ref, in_b_ref, rs_b_ref,
          o_ref,
          ws_ref, wz_s, wrs_s, bz_s, acts_ref,
          *, seqs, chans, stack, taps, folds, dilation_rate):
    G, C, S, K, F = seqs, chans, stack, taps, folds
    R = G * C
    E = 16
    T = x_ref.shape[-1]

    @pl.when(pl.program_id(0) == 0)
    def _prep():
        p_row = jax.lax.broadcasted_iota(jnp.int32, (R, C), 0) % C
        p_col = jax.lax.broadcasted_iota(jnp.int32, (R, C), 1)
        P = (p_row == p_col).astype(jnp.float32)
        q_row = jax.lax.broadcasted_iota(jnp.int32, (C, R), 0)
        q_col = jax.lax.broadcasted_iota(jnp.int32, (C, R), 1) % C
        Q = (q_row == q_col).astype(jnp.float32)
        blk = (jax.lax.broadcasted_iota(jnp.int32, (R, R), 0) // C ==
               jax.lax.broadcasted_iota(jnp.int32, (R, R), 1) // C)

        def bd(w):
            tiled = jnp.dot(jnp.dot(P, w, preferred_element_type=jnp.float32),
                            Q, preferred_element_type=jnp.float32)
            return jnp.where(blk, tiled, 0.0).astype(jnp.bfloat16)

        def tile_b(b):
            return jnp.dot(P, b, preferred_element_type=jnp.float32)

        acts_ref[R:R + E, :] = jnp.zeros((E, acts_ref.shape[-1]),
                                         jnp.bfloat16)
        acts_ref[R:R + 2, :] = jnp.ones((2, acts_ref.shape[-1]),
                                        jnp.bfloat16)

        for i in range(S):
            for h in range(2):
                r0, r1 = h * R, (h + 1) * R
                for k in range(K):
                    wz_s[i, r0:r1, k * R:(k + 1) * R] = bd(
                        in_w_ref[i, k, h * C:(h + 1) * C, :])
                wz_s[i, r0:r1, K * R:(K + 1) * R] = bd(
                    sw_ref[i, h * C:(h + 1) * C, :])
                wrs_s[i, r0:r1, 0:R] = bd(rs_w_ref[i, h * C:(h + 1) * C, :])
                wrs_s[i, r0:r1, R:R + E] = jnp.zeros((R, E), jnp.bfloat16)
                b32 = tile_b(rs_b_ref[i, h * C:(h + 1) * C, :])
                b_hi = b32.astype(jnp.bfloat16)
                wrs_s[i, r0:r1, R:R + 1] = b_hi
                wrs_s[i, r0:r1, R + 1:R + 2] = (
                    b32 - b_hi.astype(jnp.float32)).astype(jnp.bfloat16)
                bz_s[i, r0:r1, :] = tile_b(in_b_ref[i, h * C:(h + 1) * C, :])

    xs = x_ref[...]
    ms = m_ref[...]
    lane = jax.lax.broadcasted_iota(jnp.int32, (R, T), 1)

    xv = [xs[j * G:(j + 1) * G].reshape(R, T) for j in range(F)]
    mask_b = [
        jnp.broadcast_to(ms[j * G:(j + 1) * G], (G, C, T)).reshape(R, T)
        for j in range(F)]

    cs = c_ref[...]
    for j in range(F):
        ws_ref[3 * R:4 * R, j * T:(j + 1) * T] = (
            cs[j * G:(j + 1) * G].reshape(R, T).astype(jnp.bfloat16))

    skip = None
    for i in range(S):
        d = dilation_rate ** i
        for j in range(F):
            cols = slice(j * T, (j + 1) * T)
            ws_ref[0:R, cols] = jnp.where(
                lane >= d, pltpu.roll(xv[j], d, axis=1),
                0.0).astype(jnp.bfloat16)
            ws_ref[R:2 * R, cols] = xv[j].astype(jnp.bfloat16)
            ws_ref[2 * R:3 * R, cols] = jnp.where(
                lane < T - d, pltpu.roll(xv[j], T - d, axis=1),
                0.0).astype(jnp.bfloat16)
        z = (jnp.dot(wz_s[i], ws_ref[...], preferred_element_type=jnp.float32)
             + bz_s[i])
        acts_ref[0:R, :] = (jnp.tanh(z[0:R])
                            * jax.nn.sigmoid(z[R:2 * R])).astype(jnp.bfloat16)
        ro = jnp.dot(wrs_s[i], acts_ref[...],
                     preferred_element_type=jnp.float32)
        for j in range(F):
            xv[j] = (xv[j] + ro[0:R, j * T:(j + 1) * T]) * mask_b[j]
        skip = ro[R:2 * R] if i == 0 else skip + ro[R:2 * R]

    for j in range(F):
        o_ref[j * G:(j + 1) * G] = (
            skip[:, j * T:(j + 1) * T] * mask_b[j]).reshape(G, C, T).astype(
                o_ref.dtype)


def kernel(x, conditions, float_masks, style_w, in_b, in_w, rs_w, rs_b):
    B, C, T = x.shape
    Cs = conditions.shape[1]
    S, K = in_w.shape[0], in_w.shape[1]
    assert Cs == C
    dilation_rate = 2

    G, F = 16, 4
    while F > 1 and B % (G * F):
        F //= 2
    while B % (G * F):
        G //= 2
    R = G * C
    num_blocks = B // (G * F)

    sw = style_w.reshape(S, 2 * C, Cs)

    body = functools.partial(
        _body, seqs=G, chans=C, stack=S, taps=K, folds=F,
        dilation_rate=dilation_rate)

    const = lambda *shape: (shape, lambda b: (0,) * len(shape))

    out = pl.pallas_call(
        body,
        out_shape=jax.ShapeDtypeStruct((B, C, T), jnp.float32),
        grid=(num_blocks,),
        in_specs=[
            pl.BlockSpec((G * F, C, T), lambda b: (b, 0, 0)),
            pl.BlockSpec((G * F, C, T), lambda b: (b, 0, 0)),
            pl.BlockSpec((G * F, 1, T), lambda b: (b, 0, 0)),
            pl.BlockSpec(*const(S, K, 2 * C, C)),
            pl.BlockSpec(*const(S, 2 * C, Cs)),
            pl.BlockSpec(*const(S, 2 * C, C)),
            pl.BlockSpec(*const(S, 2 * C, 1)),
            pl.BlockSpec(*const(S, 2 * C, 1)),
        ],
        out_specs=pl.BlockSpec((G * F, C, T), lambda b: (b, 0, 0)),
        scratch_shapes=[
            pltpu.VMEM(((K + 1) * R, F * T), jnp.bfloat16),
            pltpu.VMEM((S, 2 * R, (K + 1) * R), jnp.bfloat16),
            pltpu.VMEM((S, 2 * R, R + 16), jnp.bfloat16),
            pltpu.VMEM((S, 2 * R, 1), jnp.float32),
            pltpu.VMEM((R + 16, F * T), jnp.bfloat16),
        ],
        compiler_params=pltpu.CompilerParams(
            dimension_semantics=("arbitrary",),
            vmem_limit_bytes=56 * 1024 * 1024),
    )(x, conditions, float_masks, in_w, sw, rs_w, in_b, rs_b)

    return out


# speedup vs baseline: 1.0306x; 1.0118x over previous
import functools

import jax
import jax.numpy as jnp
from jax.experimental import pallas as pl
from jax.experimental.pallas import tpu as pltpu


def _body(x_ref, c_ref, m_ref,
          in_w_ref, sw_ref, rs_w_ref, in_b_ref, rs_b_ref,
          o_ref,
          ws_ref, wz_s, wrs_s, bz_s, acts_ref,
          *, seqs, chans, stack, taps, folds, dilation_rate):
    G, C, S, K, F = seqs, chans, stack, taps, folds
    R = G * C
    E = 16
    T = x_ref.shape[-1]

    @pl.when(pl.program_id(0) == 0)
    def _prep():
        p_row = jax.lax.broadcasted_iota(jnp.int32, (R, C), 0) % C
        p_col = jax.lax.broadcasted_iota(jnp.int32, (R, C), 1)
        P = (p_row == p_col).astype(jnp.float32)
        q_row = jax.lax.broadcasted_iota(jnp.int32, (C, R), 0)
        q_col = jax.lax.broadcasted_iota(jnp.int32, (C, R), 1) % C
        Q = (q_row == q_col).astype(jnp.float32)
        blk = (jax.lax.broadcasted_iota(jnp.int32, (R, R), 0) // C ==
               jax.lax.broadcasted_iota(jnp.int32, (R, R), 1) // C)

        def bd(w):
            tiled = jnp.dot(jnp.dot(P, w, preferred_element_type=jnp.float32),
                            Q, preferred_element_type=jnp.float32)
            return jnp.where(blk, tiled, 0.0).astype(jnp.bfloat16)

        def tile_b(b):
            return jnp.dot(P, b, preferred_element_type=jnp.float32)

        acts_ref[R:R + E, :] = jnp.zeros((E, acts_ref.shape[-1]),
                                         jnp.bfloat16)
        acts_ref[R:R + 2, :] = jnp.ones((2, acts_ref.shape[-1]),
                                        jnp.bfloat16)

        for i in range(S):
            for h in range(2):
                r0, r1 = h * R, (h + 1) * R
                for k in range(K):
                    wz_s[i, r0:r1, k * R:(k + 1) * R] = bd(
                        in_w_ref[i, k, h * C:(h + 1) * C, :])
                wz_s[i, r0:r1, K * R:(K + 1) * R] = bd(
                    sw_ref[i, h * C:(h + 1) * C, :])
                wrs_s[i, r0:r1, 0:R] = bd(rs_w_ref[i, h * C:(h + 1) * C, :])
                wrs_s[i, r0:r1, R:R + E] = jnp.zeros((R, E), jnp.bfloat16)
                b32 = tile_b(rs_b_ref[i, h * C:(h + 1) * C, :])
                b_hi = b32.astype(jnp.bfloat16)
                wrs_s[i, r0:r1, R:R + 1] = b_hi
                wrs_s[i, r0:r1, R + 1:R + 2] = (
                    b32 - b_hi.astype(jnp.float32)).astype(jnp.bfloat16)
                bz_s[i, r0:r1, :] = tile_b(in_b_ref[i, h * C:(h + 1) * C, :])

    xs = x_ref[...]
    ms = m_ref[...]
    lane = jax.lax.broadcasted_iota(jnp.int32, (R, T), 1)

    xv = [xs[j * G:(j + 1) * G].reshape(R, T) for j in range(F)]
    mask_b = [
        jnp.broadcast_to(ms[j * G:(j + 1) * G], (G, C, T)).reshape(R, T)
        for j in range(F)]

    cs = c_ref[...]
    for j in range(F):
        ws_ref[3 * R:4 * R, j * T:(j + 1) * T] = (
            cs[j * G:(j + 1) * G].reshape(R, T).astype(jnp.bfloat16))

    skip = None
    for i in range(S):
        d = dilation_rate ** i
        for j in range(F):
            cols = slice(j * T, (j + 1) * T)
            ws_ref[0:R, cols] = jnp.where(
                lane >= d, pltpu.roll(xv[j], d, axis=1),
                0.0).astype(jnp.bfloat16)
            ws_ref[R:2 * R, cols] = xv[j].astype(jnp.bfloat16)
            ws_ref[2 * R:3 * R, cols] = jnp.where(
                lane < T - d, pltpu.roll(xv[j], T - d, axis=1),
                0.0).astype(jnp.bfloat16)
        z = (jnp.dot(wz_s[i], ws_ref[...], preferred_element_type=jnp.float32)
             + bz_s[i])
        acts_ref[0:R, :] = (jnp.tanh(z[0:R])
                            * (0.5 + 0.5 * jnp.tanh(0.5 * z[R:2 * R]))
                            ).astype(jnp.bfloat16)
        ro = jnp.dot(wrs_s[i], acts_ref[...],
                     preferred_element_type=jnp.float32)
        for j in range(F):
            xv[j] = (xv[j] + ro[0:R, j * T:(j + 1) * T]) * mask_b[j]
        skip = ro[R:2 * R] if i == 0 else skip + ro[R:2 * R]

    for j in range(F):
        o_ref[j * G:(j + 1) * G] = (
            skip[:, j * T:(j + 1) * T] * mask_b[j]).reshape(G, C, T).astype(
                o_ref.dtype)


def kernel(x, conditions, float_masks, style_w, in_b, in_w, rs_w, rs_b):
    B, C, T = x.shape
    Cs = conditions.shape[1]
    S, K = in_w.shape[0], in_w.shape[1]
    assert Cs == C
    dilation_rate = 2

    G, F = 16, 4
    while F > 1 and B % (G * F):
        F //= 2
    while B % (G * F):
        G //= 2
    R = G * C
    num_blocks = B // (G * F)

    sw = style_w.reshape(S, 2 * C, Cs)

    body = functools.partial(
        _body, seqs=G, chans=C, stack=S, taps=K, folds=F,
        dilation_rate=dilation_rate)

    const = lambda *shape: (shape, lambda b: (0,) * len(shape))

    out = pl.pallas_call(
        body,
        out_shape=jax.ShapeDtypeStruct((B, C, T), jnp.float32),
        grid=(num_blocks,),
        in_specs=[
            pl.BlockSpec((G * F, C, T), lambda b: (b, 0, 0)),
            pl.BlockSpec((G * F, C, T), lambda b: (b, 0, 0)),
            pl.BlockSpec((G * F, 1, T), lambda b: (b, 0, 0)),
            pl.BlockSpec(*const(S, K, 2 * C, C)),
            pl.BlockSpec(*const(S, 2 * C, Cs)),
            pl.BlockSpec(*const(S, 2 * C, C)),
            pl.BlockSpec(*const(S, 2 * C, 1)),
            pl.BlockSpec(*const(S, 2 * C, 1)),
        ],
        out_specs=pl.BlockSpec((G * F, C, T), lambda b: (b, 0, 0)),
        scratch_shapes=[
            pltpu.VMEM(((K + 1) * R, F * T), jnp.bfloat16),
            pltpu.VMEM((S, 2 * R, (K + 1) * R), jnp.bfloat16),
            pltpu.VMEM((S, 2 * R, R + 16), jnp.bfloat16),
            pltpu.VMEM((S, 2 * R, 1), jnp.float32),
            pltpu.VMEM((R + 16, F * T), jnp.bfloat16),
        ],
        compiler_params=pltpu.CompilerParams(
            dimension_semantics=("arbitrary",),
            vmem_limit_bytes=56 * 1024 * 1024),
    )(x, conditions, float_masks, in_w, sw, rs_w, in_b, rs_b)

    return out
